# baseline (speedup 1.0000x reference)
import functools

import jax
import jax.numpy as jnp
import numpy as _np
from jax import lax
from jax.experimental import pallas as pl
from jax.experimental.pallas import tpu as pltpu
from jax.experimental.pallas import tpu_sc as plsc

HASH_MULT = 131

_NC = 2
_NS = 16
_NW = _NC * _NS

_K = 32


def _make_sc_gather(n_total, phase_off, n_tokens, ngram_vocab, dim):
    pw = n_tokens // _NW
    nstep = pw // _K
    ngrp = dim // 32
    assert pw * _NW == n_tokens and nstep * _K == pw and nstep % 2 == 0

    mesh = plsc.VectorSubcoreMesh(core_axis_name="c", subcore_axis_name="s")

    @functools.partial(
        pl.kernel,
        mesh=mesh,
        compiler_params=pltpu.CompilerParams(needs_layout_passes=False),
        out_type=[
            jax.ShapeDtypeStruct((n_tokens, dim // 2), jnp.uint32),
            jax.ShapeDtypeStruct((n_tokens, dim // 2), jnp.uint32),
        ],
        scratch_types=[
            pltpu.VMEM((pw,), jnp.int32),
            pltpu.VMEM((pw,), jnp.int32),
            pltpu.VMEM((_K, dim), jnp.float32),
            pltpu.VMEM((_K, dim), jnp.float32),
            pltpu.VMEM((_K, dim // 2), jnp.uint32),
            pltpu.VMEM((_K, dim // 2), jnp.uint32),
            pltpu.SemaphoreType.DMA,
            pltpu.SemaphoreType.DMA,
            pltpu.SemaphoreType.DMA,
            pltpu.SemaphoreType.DMA,
        ],
    )
    def sc_gather(ids_hbm, prev_hbm, uni_hbm, ngr_hbm, uni_out, ngr_out,
                  ids_v, hsh_v, fin0, fin1, bout0, bout1, g0, g1, o0, o1):
        wid = lax.axis_index("s") * _NC + lax.axis_index("c")
        base = wid * pw
        src = phase_off + base

        pltpu.sync_copy(ids_hbm.at[pl.ds(src, pw)], ids_v)
        pltpu.sync_copy(prev_hbm.at[pl.ds(src, pw)], hsh_v)

        def hash_body(i, _):
            pv = hsh_v[pl.ds(i * 16, 16)]
            iv = ids_v[pl.ds(i * 16, 16)]
            hsh_v[pl.ds(i * 16, 16)] = lax.rem(pv * HASH_MULT + iv,
                                               ngram_vocab)
            return 0

        lax.fori_loop(0, pw // 16, hash_body, 0)

        def conv(fin, bout):
            hi_mask = jnp.uint32(0xFFFF0000)

            @plsc.parallel_loop(0, _K, unroll=4)
            def row_body(r):
                for g in range(ngrp):
                    a = fin[r, pl.ds(g * 32, 16)]
                    c = fin[r, pl.ds(g * 32 + 16, 16)]
                    au = plsc.bitcast(a, jnp.uint32)
                    cu = plsc.bitcast(c, jnp.uint32)
                    bout[r, pl.ds(g * 16, 16)] = (
                        (au >> jnp.uint32(16)) | (cu & hi_mask))

        def do_table(table_hbm, idx_ref, out_hbm):
            fin = (fin0, fin1)
            bout = (bout0, bout1)
            gsem = (g0, g1)
            osem = (o0, o1)

            def gather(c, p):
                return pltpu.async_copy(
                    table_hbm.at[idx_ref.at[pl.ds(c * _K, _K)]],
                    fin[p], gsem[p])

            def write(c, p):
                return pltpu.async_copy(
                    bout[p], out_hbm.at[pl.ds(base + c * _K, _K)],
                    osem[p])

            gather(0, 0)
            gather(1, 1)

            def step2(s2, _):
                c0 = s2 * 2
                for p in (0, 1):
                    c = c0 + p
                    pltpu.make_async_copy(
                        table_hbm.at[idx_ref.at[pl.ds(c * _K, _K)]],
                        fin[p], gsem[p]).wait()

                    @pl.when(s2 > 0)
                    def _():
                        pltpu.make_async_copy(
                            bout[p],
                            out_hbm.at[pl.ds(base + (c - 2) * _K, _K)],
                            osem[p]).wait()

                    conv(fin[p], bout[p])

                    @pl.when(c + 2 < nstep)
                    def _():
                        gather(c + 2, p)

                    write(c, p)
                return 0

            lax.fori_loop(0, nstep // 2, step2, 0)
            for p in (0, 1):
                pltpu.make_async_copy(
                    bout[p],
                    out_hbm.at[pl.ds(base + (nstep - 2 + p) * _K, _K)],
                    osem[p]).wait()

        do_table(uni_hbm, ids_v, uni_out)
        do_table(ngr_hbm, hsh_v, ngr_out)

    return sc_gather


_HI = 0xFFFF0000


def _unpack_halves(x_u32):
    a = lax.bitcast_convert_type(x_u32 << jnp.uint32(16), jnp.float32)
    c = lax.bitcast_convert_type(x_u32 & jnp.uint32(_HI), jnp.float32)
    return a, c


def _proj_body(uni_ref, ngr_ref, w_ref, b_ref, nw_ref, out_ref):
    ua, ub = _unpack_halves(uni_ref[...])
    ga, gb = _unpack_halves(ngr_ref[...])
    f = jnp.bfloat16
    x = jnp.concatenate(
        [ua.astype(f), ub.astype(f), ga.astype(f), gb.astype(f)], axis=1)
    acc = jnp.dot(x, w_ref[...], preferred_element_type=jnp.float32)
    acc = acc + b_ref[...]
    var = jnp.mean(acc * acc, axis=-1, keepdims=True)
    out_ref[...] = acc * lax.rsqrt(var + 1e-6) * nw_ref[...]


def _tc_project(uni_p, ngr_p, w_all, b, nw, block_rows=2048):
    n, h = uni_p.shape
    d = h * 2
    grid = n // block_rows
    assert grid * block_rows == n
    return pl.pallas_call(
        _proj_body,
        grid=(grid,),
        in_specs=[
            pl.BlockSpec((block_rows, h), lambda i: (i, 0)),
            pl.BlockSpec((block_rows, h), lambda i: (i, 0)),
            pl.BlockSpec((2 * d, d), lambda i: (0, 0)),
            pl.BlockSpec((1, d), lambda i: (0, 0)),
            pl.BlockSpec((1, d), lambda i: (0, 0)),
        ],
        out_specs=pl.BlockSpec((block_rows, d), lambda i: (i, 0)),
        out_shape=jax.ShapeDtypeStruct((n, d), jnp.float32),
    )(uni_p, ngr_p, w_all, b, nw)


_N_PHASES = 1


def kernel(input_ids, unigram_table, ngram_table, W, b, norm_weight):
    bb, ss = input_ids.shape
    vocab, dim = unigram_table.shape
    ngram_vocab = ngram_table.shape[0]
    n = bb * ss

    ids = input_ids.reshape(n).astype(jnp.int32)
    prev = jnp.pad(input_ids, ((0, 0), (1, 0)))[:, :-1].reshape(n)
    prev = prev.astype(jnp.int32)

    wt = W.T
    wi = _np.arange(dim // 2)
    pa = (wi // 16) * 32 + wi % 16
    pb = pa + 16
    w1, w2 = wt[:dim], wt[dim:]
    w_all = jnp.concatenate(
        [w1[pa], w1[pb], w2[pa], w2[pb]], axis=0).astype(jnp.bfloat16)
    b2 = b.reshape(1, dim)
    nw2 = norm_weight.reshape(1, dim)

    np_ = n // _N_PHASES
    outs = []
    for p in range(_N_PHASES):
        sc_gather = _make_sc_gather(n, p * np_, np_, ngram_vocab, dim)
        u, g = sc_gather(ids, prev, unigram_table, ngram_table)
        outs.append(_tc_project(u, g, w_all, b2, nw2))
    out = jnp.concatenate(outs, axis=0)
    return out.reshape(bb, ss, dim)
